# Initial kernel scaffold; baseline (speedup 1.0000x reference)
import jax
import jax.numpy as jnp
from jax.experimental import pallas as pl


def kernel(seq_len, relative_position_bias_table):
    raise NotImplementedError("write your pallas kernel here")


# speedup vs baseline: 9.8506x; 9.8506x over previous
import jax
import jax.numpy as jnp
from jax import lax
from jax.experimental import pallas as pl
from jax.experimental.pallas import tpu as pltpu
from jax.experimental.pallas import tpu_sc as plsc

_SEQ = 2048
_H = 16
_NC = 2
_NS = 16
_NW = _NC * _NS
_RPW = _SEQ // _NW
_STAGE = _SEQ + _RPW
_ROW_W = _SEQ * _H


def _rpe_body(table_hbm, out_hbm, buf_v, sem):
    cid = lax.axis_index("c")
    sid = lax.axis_index("s")
    wid = sid * _NC + cid
    base = wid * _RPW

    pltpu.sync_copy(
        table_hbm.at[pl.ds((_SEQ + base) * _H, _STAGE * _H)], buf_v
    )

    def _rev(g, carry):
        for u in range(8):
            t = g * 8 + u
            lo = buf_v[pl.ds(t * _H, _H)]
            hi = buf_v[pl.ds((_STAGE - 1 - t) * _H, _H)]
            buf_v[pl.ds(t * _H, _H)] = hi
            buf_v[pl.ds((_STAGE - 1 - t) * _H, _H)] = lo
        return carry

    lax.fori_loop(0, _STAGE // 2 // 8, _rev, 0)

    def _wr(g, carry):
        handles = []
        for u in range(8):
            r = g * 8 + u
            handles.append(
                pltpu.async_copy(
                    buf_v.at[pl.ds((_RPW - r) * _H, _ROW_W)],
                    out_hbm.at[pl.ds((base + r) * _ROW_W, _ROW_W)],
                    sem,
                )
            )
        for h in handles:
            h.wait()
        return carry

    lax.fori_loop(0, _RPW // 8, _wr, 0)


@jax.jit
def _rpe(table):
    mesh = plsc.VectorSubcoreMesh(core_axis_name="c", subcore_axis_name="s")
    fn = pl.kernel(
        _rpe_body,
        out_type=jax.ShapeDtypeStruct((_SEQ * _SEQ * _H,), jnp.float32),
        mesh=mesh,
        scratch_types=[
            pltpu.VMEM((_STAGE * _H,), jnp.float32),
            pltpu.SemaphoreType.DMA,
        ],
    )
    out = fn(table.reshape(-1))
    return out.reshape(_SEQ, _SEQ, _H)


def kernel(seq_len, relative_position_bias_table):
    del seq_len
    return _rpe(relative_position_bias_table)
